# baseline (speedup 1.0000x reference)
import functools

import jax
import jax.numpy as jnp
from jax import lax
from jax.experimental import pallas as pl
from jax.experimental.pallas import tpu as pltpu
from jax.experimental.pallas import tpu_sc as plsc

F = 26
V = 100000
D = 32
B = 16384

NW = 32
N_TOTAL = B * F
PER_W = N_TOTAL // NW
G = 128
NG = PER_W // G
CHUNK_G = 8
NCHUNK = NG // CHUNK_G
CHUNK_ROWS = CHUNK_G * G
L = 16


def _body(tab_hbm, x_hbm, out_hbm, idx_v, rows_v, sem):
    cid = lax.axis_index("c")
    sid = lax.axis_index("s")
    wid = sid * 2 + cid
    base = wid * PER_W

    pltpu.sync_copy(x_hbm.at[pl.ds(wid * NG, NG)], idx_v)

    def off_body(t, _):
        g = t // (G // L)
        s = t % (G // L)
        pos = t * L + lax.iota(jnp.int32, L)
        offs = lax.rem(pos, F) * V
        idx_v[g, pl.ds(s * L, L)] = idx_v[g, pl.ds(s * L, L)] + offs
        return _

    lax.fori_loop(0, NG * (G // L), off_body, 0)

    def chunk_body(c, _):
        g0 = c * CHUNK_G
        cps = []
        for s in range(CHUNK_G):
            cp = pltpu.async_copy(
                tab_hbm.at[idx_v.at[g0 + s]],
                rows_v.at[pl.ds(s * G, G)],
                sem,
            )
            cps.append(cp)
        for cp in cps:
            cp.wait()
        pltpu.sync_copy(rows_v, out_hbm.at[pl.ds(base + c * CHUNK_ROWS, CHUNK_ROWS)])
        return _

    lax.fori_loop(0, NCHUNK, chunk_body, 0)


@jax.jit
def _run(x_flat, tab_flat):
    mesh = plsc.VectorSubcoreMesh(core_axis_name="c", subcore_axis_name="s")
    kfn = pl.kernel(
        _body,
        mesh=mesh,
        out_type=jax.ShapeDtypeStruct((N_TOTAL, D), jnp.float32),
        scratch_types=[
            pltpu.VMEM((NG, G), jnp.int32),
            pltpu.VMEM((CHUNK_ROWS, D), jnp.float32),
            pltpu.SemaphoreType.DMA,
        ],
        compiler_params=pltpu.CompilerParams(use_tc_tiling_on_sc=False),
    )
    return kfn(tab_flat, x_flat)


def kernel(x, tables):
    x_flat = x.astype(jnp.int32).reshape(N_TOTAL // G, G)
    tab_flat = tables.reshape(F * V, D)
    out = _run(x_flat, tab_flat)
    return out.reshape(B, F * D)


# speedup vs baseline: 1.0062x; 1.0062x over previous
import jax
import jax.numpy as jnp
from jax import lax
from jax.experimental import pallas as pl
from jax.experimental.pallas import tpu as pltpu
from jax.experimental.pallas import tpu_sc as plsc

F = 26
V = 100000
D = 32
B = 16384

NW = 32
N_TOTAL = B * F
PER_W = N_TOTAL // NW
G = 512
NG = PER_W // G
L = 16


def _body(tab_hbm, x_hbm, out_hbm, idx_v, rows_v, g0, g1, w0, w1):
    cid = lax.axis_index("c")
    sid = lax.axis_index("s")
    wid = sid * 2 + cid
    base = wid * PER_W
    sem_g = [g0, g1]
    sem_w = [w0, w1]

    pltpu.sync_copy(x_hbm.at[pl.ds(wid * NG, NG)], idx_v)

    def off_body(t, carry):
        g = t // (G // L)
        s = t % (G // L)
        pos = t * L + lax.iota(jnp.int32, L)
        offs = lax.rem(pos, F) * V
        idx_v[g, pl.ds(s * L, L)] = idx_v[g, pl.ds(s * L, L)] + offs
        return carry

    lax.fori_loop(0, NG * (G // L), off_body, 0)

    def fire_gather(c, b):
        pltpu.async_copy(tab_hbm.at[idx_v.at[c]], rows_v.at[b], sem_g[b])

    def wait_gather(c, b):
        pltpu.make_async_copy(tab_hbm.at[idx_v.at[c]], rows_v.at[b], sem_g[b]).wait()

    def fire_write(c, b):
        pltpu.async_copy(rows_v.at[b], out_hbm.at[pl.ds(base + c * G, G)], sem_w[b])

    def wait_write(c, b):
        pltpu.make_async_copy(rows_v.at[b], out_hbm.at[pl.ds(base + c * G, G)], sem_w[b]).wait()

    fire_gather(0, 0)
    fire_gather(1, 1)

    def outer(i, carry):
        for b in range(2):
            c = i * 2 + b
            wait_gather(c, b)
            fire_write(c, b)

            @pl.when(c + 2 < NG)
            def _():
                wait_write(c, b)
                fire_gather(c + 2, b)

        return carry

    lax.fori_loop(0, NG // 2, outer, 0)

    wait_write(NG - 2, 0)
    wait_write(NG - 1, 1)


@jax.jit
def _run(x_flat, tab_flat):
    mesh = plsc.VectorSubcoreMesh(core_axis_name="c", subcore_axis_name="s")
    kfn = pl.kernel(
        _body,
        mesh=mesh,
        out_type=jax.ShapeDtypeStruct((N_TOTAL, D), jnp.float32),
        scratch_types=[
            pltpu.VMEM((NG, G), jnp.int32),
            pltpu.VMEM((2, G, D), jnp.float32),
            pltpu.SemaphoreType.DMA,
            pltpu.SemaphoreType.DMA,
            pltpu.SemaphoreType.DMA,
            pltpu.SemaphoreType.DMA,
        ],
        compiler_params=pltpu.CompilerParams(use_tc_tiling_on_sc=False),
    )
    return kfn(tab_flat, x_flat)


def kernel(x, tables):
    x_flat = x.astype(jnp.int32).reshape(NW * NG, G)
    tab_flat = tables.reshape(F * V, D)
    out = _run(x_flat, tab_flat)
    return out.reshape(B, F * D)


# speedup vs baseline: 4.0052x; 3.9805x over previous
import jax
import jax.numpy as jnp
from jax import lax
from jax.experimental import pallas as pl
from jax.experimental.pallas import tpu as pltpu
from jax.experimental.pallas import tpu_sc as plsc

F = 26
V = 100000
D = 32
B = 16384

NW = 32
NTASK = F * D
PER_W = NTASK // NW
CHUNK = 8192
NCHUNK = B // CHUNK
L = 16
UNROLL = 8


def _body(tab_hbm, x_hbm, out_hbm, row_v, idx_v, out_v):
    cid = lax.axis_index("c")
    sid = lax.axis_index("s")
    wid = sid * 2 + cid

    def task_body(j, carry):
        task = wid * PER_W + j
        f = task // D
        c = task % D

        pltpu.sync_copy(tab_hbm.at[f, c], row_v)

        def chunk_body(k, carry2):
            b0 = k * CHUNK
            pltpu.sync_copy(x_hbm.at[f, pl.ds(b0, CHUNK)], idx_v)

            def gather_body(g, carry3):
                base = g * (L * UNROLL)
                for u in range(UNROLL):
                    off = base + u * L
                    iv = idx_v[pl.ds(off, L)]
                    out_v[pl.ds(off, L)] = plsc.load_gather(row_v, [iv])
                return carry3

            lax.fori_loop(0, CHUNK // (L * UNROLL), gather_body, 0)
            pltpu.sync_copy(out_v, out_hbm.at[task, pl.ds(b0, CHUNK)])
            return carry2

        lax.fori_loop(0, NCHUNK, chunk_body, 0)
        return carry

    lax.fori_loop(0, PER_W, task_body, 0)


@jax.jit
def _run(x_t, tab_t):
    mesh = plsc.VectorSubcoreMesh(core_axis_name="c", subcore_axis_name="s")
    kfn = pl.kernel(
        _body,
        mesh=mesh,
        out_type=jax.ShapeDtypeStruct((NTASK, B), jnp.float32),
        scratch_types=[
            pltpu.VMEM((V,), jnp.float32),
            pltpu.VMEM((CHUNK,), jnp.int32),
            pltpu.VMEM((CHUNK,), jnp.float32),
        ],
        compiler_params=pltpu.CompilerParams(
            use_tc_tiling_on_sc=True, needs_layout_passes=False
        ),
    )
    return kfn(tab_t, x_t)


def kernel(x, tables):
    x_t = x.astype(jnp.int32).T
    tab_t = jnp.transpose(tables, (0, 2, 1))
    out_t = _run(x_t, tab_t)
    return out_t.T
